# baseline (speedup 1.0000x reference)
import functools

import jax
import jax.numpy as jnp
from jax import lax
from jax.experimental import pallas as pl
from jax.experimental.pallas import tpu as pltpu
from jax.experimental.pallas import tpu_sc as plsc

NC = 2
NS = 16
NW = NC * NS


def _combine_table(ww2, wp2, W_tok, hidden):
    def body(ww, wp, wt, o):
        tt = ww[...] + wp[...] + wt[...]
        o[...] = jnp.concatenate(
            [tt, jnp.zeros((6, tt.shape[1]), jnp.float32)], axis=0
        )

    return pl.pallas_call(
        body,
        grid=(1,),
        out_shape=jax.ShapeDtypeStruct((8, hidden), jnp.float32),
        in_specs=[
            pl.BlockSpec((2, hidden), lambda i: (0, 0)),
            pl.BlockSpec((2, hidden), lambda i: (0, 0)),
            pl.BlockSpec((2, hidden), lambda i: (0, 0)),
        ],
        out_specs=pl.BlockSpec((8, hidden), lambda i: (0, 0)),
    )(ww2, wp2, W_tok)


def _sc_select(ids_t, t2, nb, seq, hidden):
    bw = nb // NW
    ng = bw // 16

    mesh = plsc.VectorSubcoreMesh(core_axis_name="c", subcore_axis_name="s")

    @functools.partial(
        pl.kernel,
        mesh=mesh,
        out_type=jax.ShapeDtypeStruct((seq, hidden, nb), jnp.float32),
        scratch_types=[
            pltpu.VMEM((seq, bw), jnp.int32),
            pltpu.VMEM((8, hidden), jnp.float32),
            pltpu.VMEM((2, hidden, 16), jnp.float32),
            pltpu.VMEM((2, 1, hidden, bw), jnp.float32),
            pltpu.SemaphoreType.DMA,
        ],
        compiler_params=pltpu.CompilerParams(use_tc_tiling_on_sc=True),
    )
    def k(ids_hbm, t_hbm, out_hbm, ids_v, t_loc, bc, rbuf, ssem):
        cid = lax.axis_index("c")
        sid = lax.axis_index("s")
        wid = sid * NC + cid
        b0 = wid * bw

        pltpu.sync_copy(t_hbm, t_loc)
        pltpu.sync_copy(ids_hbm.at[:, pl.ds(b0, bw)], ids_v)

        for r in range(2):
            for j in range(hidden // 16):
                v = t_loc[r, pl.ds(16 * j, 16)]
                for l in range(16):
                    bc[r, 16 * j + l, pl.ds(0, 16)] = jnp.full(
                        (16,), v[l], jnp.float32
                    )

        def pos_body(s, carry):
            par = s % 2

            @pl.when(s >= 2)
            def _():
                pltpu.make_async_copy(
                    rbuf.at[par], out_hbm.at[pl.ds(0, 1), :, pl.ds(b0, bw)],
                    ssem,
                ).wait()

            masks = [ids_v[s, pl.ds(16 * g, 16)] != 0 for g in range(ng)]
            for h in range(hidden):
                t0 = bc[0, h, pl.ds(0, 16)]
                t1 = bc[1, h, pl.ds(0, 16)]
                for g in range(ng):
                    rbuf[par, 0, h, pl.ds(16 * g, 16)] = jnp.where(
                        masks[g], t1, t0
                    )

            pltpu.async_copy(
                rbuf.at[par], out_hbm.at[pl.ds(s, 1), :, pl.ds(b0, bw)], ssem
            )
            return carry

        lax.fori_loop(0, seq, pos_body, 0)

        pltpu.make_async_copy(
            rbuf.at[0], out_hbm.at[pl.ds(0, 1), :, pl.ds(b0, bw)], ssem
        ).wait()
        pltpu.make_async_copy(
            rbuf.at[1], out_hbm.at[pl.ds(0, 1), :, pl.ds(b0, bw)], ssem
        ).wait()

    return k(ids_t, t2)


def kernel(input_ids, W_word, W_pos, W_tok):
    nb, seq = input_ids.shape
    hidden = W_word.shape[1]
    ids_t = input_ids.astype(jnp.int32).T
    t2 = _combine_table(W_word[0:2], W_pos[0:2], W_tok, hidden)
    y = _sc_select(ids_t, t2, nb, seq, hidden)
    return jnp.transpose(y, (2, 0, 1))


# speedup vs baseline: 1.5411x; 1.5411x over previous
import functools

import jax
import jax.numpy as jnp
from jax import lax
from jax.experimental import pallas as pl
from jax.experimental.pallas import tpu as pltpu
from jax.experimental.pallas import tpu_sc as plsc

NC = 2
NS = 16
NW = NC * NS


def _combine_table(ww2, wp2, W_tok, hidden):
    def body(ww, wp, wt, o):
        tt = ww[...] + wp[...] + wt[...]
        o[...] = jnp.concatenate(
            [tt, jnp.zeros((6, tt.shape[1]), jnp.float32)], axis=0
        )

    return pl.pallas_call(
        body,
        grid=(1,),
        out_shape=jax.ShapeDtypeStruct((8, hidden), jnp.float32),
        in_specs=[
            pl.BlockSpec((2, hidden), lambda i: (0, 0)),
            pl.BlockSpec((2, hidden), lambda i: (0, 0)),
            pl.BlockSpec((2, hidden), lambda i: (0, 0)),
        ],
        out_specs=pl.BlockSpec((8, hidden), lambda i: (0, 0)),
    )(ww2, wp2, W_tok)


def _sc_select(ids_t, t2, nb, seq, hidden):
    bw = nb // NW
    ng = bw // 16

    mesh = plsc.VectorSubcoreMesh(core_axis_name="c", subcore_axis_name="s")

    @functools.partial(
        pl.kernel,
        mesh=mesh,
        out_type=jax.ShapeDtypeStruct((seq, hidden, nb), jnp.float32),
        scratch_types=[
            pltpu.VMEM((seq, bw), jnp.int32),
            pltpu.VMEM((8, hidden), jnp.float32),
            pltpu.VMEM((4, 1, hidden, bw), jnp.float32),
            pltpu.SemaphoreType.DMA,
        ],
        compiler_params=pltpu.CompilerParams(use_tc_tiling_on_sc=True),
    )
    def k(ids_hbm, t_hbm, out_hbm, ids_v, t_loc, rbuf, ssem):
        cid = lax.axis_index("c")
        sid = lax.axis_index("s")
        wid = sid * NC + cid
        b0 = wid * bw

        pltpu.sync_copy(t_hbm, t_loc)
        pltpu.sync_copy(ids_hbm.at[:, pl.ds(b0, bw)], ids_v)

        def pos_body(s, carry):
            par = s % 4

            @pl.when(s >= 4)
            def _():
                pltpu.make_async_copy(
                    rbuf.at[par], out_hbm.at[pl.ds(0, 1), :, pl.ds(b0, bw)],
                    ssem,
                ).wait()

            masks = [ids_v[s, pl.ds(16 * g, 16)] != 0 for g in range(ng)]
            row0 = [t_loc[0, pl.ds(16 * j, 16)] for j in range(hidden // 16)]
            row1 = [t_loc[1, pl.ds(16 * j, 16)] for j in range(hidden // 16)]
            for h in range(hidden):
                t0 = row0[h // 16][h % 16]
                t1 = row1[h // 16][h % 16]
                for g in range(ng):
                    rbuf[par, 0, h, pl.ds(16 * g, 16)] = jnp.where(
                        masks[g], t1, t0
                    )

            pltpu.async_copy(
                rbuf.at[par], out_hbm.at[pl.ds(s, 1), :, pl.ds(b0, bw)], ssem
            )
            return carry

        lax.fori_loop(0, seq, pos_body, 0)

        for q in range(4):
            pltpu.make_async_copy(
                rbuf.at[q], out_hbm.at[pl.ds(0, 1), :, pl.ds(b0, bw)], ssem
            ).wait()

    return k(ids_t, t2)


def kernel(input_ids, W_word, W_pos, W_tok):
    nb, seq = input_ids.shape
    hidden = W_word.shape[1]
    ids_t = input_ids.astype(jnp.int32).T
    t2 = _combine_table(W_word[0:2], W_pos[0:2], W_tok, hidden)
    y = _sc_select(ids_t, t2, nb, seq, hidden)
    return jnp.transpose(y, (2, 0, 1))
